# Initial kernel scaffold; baseline (speedup 1.0000x reference)
import jax
import jax.numpy as jnp
from jax.experimental import pallas as pl


def kernel(x, edge_index, edge_attr, batch, W1, b1, W2, b2, W3, b3, Wc, bc):
    raise NotImplementedError("write your pallas kernel here")


# speedup vs baseline: 12.4437x; 12.4437x over previous
import functools

import jax
import jax.numpy as jnp
from jax import lax
from jax.experimental import pallas as pl
from jax.experimental.pallas import tpu as pltpu
from jax.experimental.pallas import tpu_sc as plsc

_N = 10000
_E = 320000
_DIN = 128
_H = 64
_C = 100
_B = 16

_NC = 2
_NS = 16
_NW = _NC * _NS
_SEG = 80
_BLK = 400
_NSEG = _BLK // _SEG
_EPW = _E // _NW
_NBLK = _EPW // _BLK
_RPS = _N // _NS
_HCH = 640

_mesh = plsc.VectorSubcoreMesh(
    core_axis_name="c", subcore_axis_name="s", num_cores=_NC, num_subcores=_NS
)

_sc_params = pltpu.CompilerParams(use_tc_tiling_on_sc=False)


def _repack(flat, two_d, nseg):
    for j in range(nseg):
        for q in range(_SEG // 16):
            two_d[j, pl.ds(q * 16, 16)] = flat[pl.ds(j * _SEG + q * 16, 16)]


def _hist_w_body(col_hbm, ea_hbm, hist_hbm, w_hbm, colf, colb, eab, wbuf,
                 ones, acc_sh, sem):
    c = lax.axis_index("c")
    s = lax.axis_index("s")
    wid = s * _NC + c

    @pl.loop(0, _SEG, step=16)
    def _(i):
        ones[pl.ds(i, 16)] = jnp.full((16,), 1.0, jnp.float32)

    @pl.loop(0, _BLK, step=16)
    def _(i):
        wbuf[pl.ds(i, 16)] = jnp.zeros((16,), jnp.float32)

    @pl.when(s < _NS - 1)
    def _():
        pltpu.sync_copy(wbuf, acc_sh.at[pl.ds(s * _HCH, _BLK)])
        pltpu.sync_copy(wbuf.at[pl.ds(0, _HCH - _BLK)],
                        acc_sh.at[pl.ds(s * _HCH + _BLK, _HCH - _BLK)])

    @pl.when(s == _NS - 1)
    def _():
        pltpu.sync_copy(wbuf, acc_sh.at[pl.ds(s * _HCH, _BLK)])

    plsc.subcore_barrier()

    @pl.loop(0, _NBLK)
    def _(b):
        ebase = wid * _EPW + b * _BLK
        pltpu.sync_copy(col_hbm.at[pl.ds(ebase, _BLK)], colf)
        pltpu.sync_copy(ea_hbm.at[pl.ds(ebase, _BLK)], eab)
        _repack(colf, colb, _NSEG)

        @pl.loop(0, _BLK, step=16)
        def _(i):
            wbuf[pl.ds(i, 16)] = jnp.exp(-eab[pl.ds(i, 16)])

        pltpu.sync_copy(wbuf, w_hbm.at[pl.ds(ebase, _BLK)])
        cps = [pltpu.async_copy(ones, acc_sh.at[colb.at[j]], sem, add=True)
               for j in range(_NSEG)]
        for cp in cps:
            cp.wait()

    plsc.subcore_barrier()

    @pl.when(s < _NS - 1)
    def _():
        pltpu.sync_copy(acc_sh.at[pl.ds(s * _HCH, _HCH)],
                        hist_hbm.at[c].at[pl.ds(s * _HCH, _HCH)])

    @pl.when(s == _NS - 1)
    def _():
        pltpu.sync_copy(acc_sh.at[pl.ds(s * _HCH, _BLK)],
                        hist_hbm.at[c].at[pl.ds(s * _HCH, _BLK)])


_hist_w = pl.kernel(
    _hist_w_body,
    out_type=(
        jax.ShapeDtypeStruct((_NC, _N), jnp.float32),
        jax.ShapeDtypeStruct((_E,), jnp.float32),
    ),
    mesh=_mesh,
    compiler_params=_sc_params,
    scratch_types=[
        pltpu.VMEM((_BLK,), jnp.int32),
        pltpu.VMEM((_NSEG, _SEG), jnp.int32),
        pltpu.VMEM((_BLK,), jnp.float32),
        pltpu.VMEM((_BLK,), jnp.float32),
        pltpu.VMEM((_SEG,), jnp.float32),
        pltpu.VMEM_SHARED((_N,), jnp.float32),
        pltpu.SemaphoreType.DMA,
    ],
)


def _mp_body(g_hbm, row_hbm, col_hbm, w_hbm, p_hbm, rowf, colf, colb, wb,
             gbuf, acc_sh, gsem, ssem):
    c = lax.axis_index("c")
    s = lax.axis_index("s")
    wid = s * _NC + c

    @pl.loop(0, _BLK)
    def _(r):
        for q in range(_H // 16):
            gbuf[r, pl.ds(q * 16, 16)] = jnp.zeros((16,), jnp.float32)

    pltpu.sync_copy(gbuf, acc_sh.at[pl.ds(s * _RPS, _BLK)])
    pltpu.sync_copy(gbuf.at[pl.ds(0, _RPS - _BLK)],
                    acc_sh.at[pl.ds(s * _RPS + _BLK, _RPS - _BLK)])
    plsc.subcore_barrier()

    @pl.loop(0, _NBLK)
    def _(b):
        ebase = wid * _EPW + b * _BLK
        pltpu.sync_copy(row_hbm.at[pl.ds(ebase, _BLK)], rowf)
        pltpu.sync_copy(col_hbm.at[pl.ds(ebase, _BLK)], colf)
        pltpu.sync_copy(w_hbm.at[pl.ds(ebase, _BLK)], wb)
        _repack(colf, colb, _NSEG)

        gcps = [pltpu.async_copy(g_hbm.at[rowf.at[pl.ds(j * _SEG, _SEG)]],
                                 gbuf.at[pl.ds(j * _SEG, _SEG)], gsem)
                for j in range(_NSEG)]
        for cp in gcps:
            cp.wait()

        @pl.loop(0, _BLK, step=16)
        def _(i):
            wv = wb[pl.ds(i, 16)]
            for k in range(16):
                wk = jnp.full((16,), wv[k], jnp.float32)
                for q in range(_H // 16):
                    gbuf[i + k, pl.ds(q * 16, 16)] = (
                        gbuf[i + k, pl.ds(q * 16, 16)] * wk)

        scps = [pltpu.async_copy(gbuf.at[pl.ds(j * _SEG, _SEG)],
                                 acc_sh.at[colb.at[j]], ssem, add=True)
                for j in range(_NSEG)]
        for cp in scps:
            cp.wait()

    plsc.subcore_barrier()
    pltpu.sync_copy(acc_sh.at[pl.ds(s * _RPS, _RPS)],
                    p_hbm.at[c].at[pl.ds(s * _RPS, _RPS)])


_mp = pl.kernel(
    _mp_body,
    out_type=jax.ShapeDtypeStruct((_NC, _N, _H), jnp.float32),
    mesh=_mesh,
    compiler_params=_sc_params,
    scratch_types=[
        pltpu.VMEM((_BLK,), jnp.int32),
        pltpu.VMEM((_BLK,), jnp.int32),
        pltpu.VMEM((_NSEG, _SEG), jnp.int32),
        pltpu.VMEM((_BLK,), jnp.float32),
        pltpu.VMEM((_BLK, _H), jnp.float32),
        pltpu.VMEM_SHARED((_N, _H), jnp.float32),
        pltpu.SemaphoreType.DMA,
        pltpu.SemaphoreType.DMA,
    ],
)


def _prep_body(hist_ref, x_ref, w1_ref, dis_ref, g_ref):
    hist = hist_ref[...]
    deg = hist[0] + hist[1] + 1.0
    dis = lax.rsqrt(deg)
    dis_ref[...] = jnp.reshape(dis, (_N, 1))
    xw = lax.dot_general(x_ref[...], w1_ref[...], (((1,), (1,)), ((), ())),
                         preferred_element_type=jnp.float32)
    g_ref[...] = jnp.reshape(dis, (_N, 1)) * xw


_prep = pl.pallas_call(
    _prep_body,
    out_shape=(
        jax.ShapeDtypeStruct((_N, 1), jnp.float32),
        jax.ShapeDtypeStruct((_N, _H), jnp.float32),
    ),
)


def _layer_body(p_ref, g_ref, dis_ref, b_ref, w_ref, gn_ref):
    dis = dis_ref[...]
    agg = p_ref[0] + p_ref[1] + g_ref[...]
    h = jnp.maximum(dis * agg + b_ref[...], 0.0)
    hw = lax.dot_general(h, w_ref[...], (((1,), (1,)), ((), ())),
                         preferred_element_type=jnp.float32)
    gn_ref[...] = dis * hw


_layer = pl.pallas_call(
    _layer_body,
    out_shape=jax.ShapeDtypeStruct((_N, _H), jnp.float32),
)


def _final_body(p_ref, g_ref, dis_ref, b_ref, batch_ref, wc_ref, bc_ref,
                out_ref):
    dis = dis_ref[...]
    h = jnp.maximum(dis * (p_ref[0] + p_ref[1] + g_ref[...]) + b_ref[...],
                    0.0)
    onehot = (batch_ref[...] ==
              lax.broadcasted_iota(jnp.int32, (_B, _N), 0)).astype(jnp.float32)
    sums = lax.dot_general(onehot, h, (((1,), (0,)), ((), ())),
                           preferred_element_type=jnp.float32)
    cnt = jnp.sum(onehot, axis=1, keepdims=True)
    pooled = sums / jnp.maximum(cnt, 1.0)
    out = lax.dot_general(pooled, wc_ref[...], (((1,), (1,)), ((), ())),
                          preferred_element_type=jnp.float32)
    out_ref[...] = out + bc_ref[...]


_final = pl.pallas_call(
    _final_body,
    out_shape=jax.ShapeDtypeStruct((_B, _C), jnp.float32),
)


def kernel(x, edge_index, edge_attr, batch, W1, b1, W2, b2, W3, b3, Wc, bc):
    row = edge_index[0]
    col = edge_index[1]
    hist, w = _hist_w(col, edge_attr)
    dis, g1 = _prep(hist, x, W1)
    p1 = _mp(g1, row, col, w)
    g2 = _layer(p1, g1, dis, b1.reshape(1, _H), W2)
    p2 = _mp(g2, row, col, w)
    g3 = _layer(p2, g2, dis, b2.reshape(1, _H), W3)
    p3 = _mp(g3, row, col, w)
    return _final(p3, g3, dis, b3.reshape(1, _H), batch.reshape(1, _N),
                  Wc, bc.reshape(1, _C))


# speedup vs baseline: 29.8862x; 2.4017x over previous
import functools

import jax
import jax.numpy as jnp
from jax import lax
from jax.experimental import pallas as pl
from jax.experimental.pallas import tpu as pltpu
from jax.experimental.pallas import tpu_sc as plsc

_N = 10000
_E = 320000
_DIN = 128
_H = 64
_C = 100
_B = 16

_NC = 2
_NS = 16
_NW = _NC * _NS
_SEG = 80
_BLK = 400
_NSEG = _BLK // _SEG
_EPW = _E // _NW
_NBLK = _EPW // _BLK
_RPS = _N // _NS
_HCH = 640
_NBUF = 3

_mesh = plsc.VectorSubcoreMesh(
    core_axis_name="c", subcore_axis_name="s", num_cores=_NC, num_subcores=_NS
)

_sc_params = pltpu.CompilerParams(use_tc_tiling_on_sc=False)


def _repack(flat, two_d, nseg):
    for j in range(nseg):
        for q in range(_SEG // 16):
            two_d[j, pl.ds(q * 16, 16)] = flat[pl.ds(j * _SEG + q * 16, 16)]


def _hist_w_body(col_hbm, ea_hbm, hist_hbm, w_hbm, colf, colb, eab, wbuf,
                 ones, acc_sh, sem):
    c = lax.axis_index("c")
    s = lax.axis_index("s")
    wid = s * _NC + c

    @pl.loop(0, _SEG, step=16)
    def _(i):
        ones[pl.ds(i, 16)] = jnp.full((16,), 1.0, jnp.float32)

    @pl.loop(0, _BLK, step=16)
    def _(i):
        wbuf[pl.ds(i, 16)] = jnp.zeros((16,), jnp.float32)

    @pl.when(s < _NS - 1)
    def _():
        pltpu.sync_copy(wbuf, acc_sh.at[pl.ds(s * _HCH, _BLK)])
        pltpu.sync_copy(wbuf.at[pl.ds(0, _HCH - _BLK)],
                        acc_sh.at[pl.ds(s * _HCH + _BLK, _HCH - _BLK)])

    @pl.when(s == _NS - 1)
    def _():
        pltpu.sync_copy(wbuf, acc_sh.at[pl.ds(s * _HCH, _BLK)])

    plsc.subcore_barrier()

    @pl.loop(0, _NBLK)
    def _(b):
        ebase = wid * _EPW + b * _BLK
        pltpu.sync_copy(col_hbm.at[pl.ds(ebase, _BLK)], colf)
        pltpu.sync_copy(ea_hbm.at[pl.ds(ebase, _BLK)], eab)
        _repack(colf, colb, _NSEG)

        @pl.loop(0, _BLK, step=16)
        def _(i):
            wbuf[pl.ds(i, 16)] = jnp.exp(-eab[pl.ds(i, 16)])

        pltpu.sync_copy(wbuf, w_hbm.at[pl.ds(ebase, _BLK)])
        cps = [pltpu.async_copy(ones, acc_sh.at[colb.at[j]], sem, add=True)
               for j in range(_NSEG)]
        for cp in cps:
            cp.wait()

    plsc.subcore_barrier()

    @pl.when(s < _NS - 1)
    def _():
        pltpu.sync_copy(acc_sh.at[pl.ds(s * _HCH, _HCH)],
                        hist_hbm.at[c].at[pl.ds(s * _HCH, _HCH)])

    @pl.when(s == _NS - 1)
    def _():
        pltpu.sync_copy(acc_sh.at[pl.ds(s * _HCH, _BLK)],
                        hist_hbm.at[c].at[pl.ds(s * _HCH, _BLK)])


_hist_w = pl.kernel(
    _hist_w_body,
    out_type=(
        jax.ShapeDtypeStruct((_NC, _N), jnp.float32),
        jax.ShapeDtypeStruct((_E,), jnp.float32),
    ),
    mesh=_mesh,
    compiler_params=_sc_params,
    scratch_types=[
        pltpu.VMEM((_BLK,), jnp.int32),
        pltpu.VMEM((_NSEG, _SEG), jnp.int32),
        pltpu.VMEM((_BLK,), jnp.float32),
        pltpu.VMEM((_BLK,), jnp.float32),
        pltpu.VMEM((_SEG,), jnp.float32),
        pltpu.VMEM_SHARED((_N,), jnp.float32),
        pltpu.SemaphoreType.DMA,
    ],
)


def _mp_body(g_hbm, row_hbm, col_hbm, w_hbm, p_hbm,
             rowf0, colb0, wb0, gbuf0,
             rowf1, colb1, wb1, gbuf1,
             rowf2, colb2, wb2, gbuf2,
             colf, acc_sh, gsem0, gsem1, gsem2, ssem0, ssem1, ssem2):
    c = lax.axis_index("c")
    s = lax.axis_index("s")
    wid = s * _NC + c
    e0 = wid * _EPW
    bufs = ((rowf0, colb0, wb0, gbuf0, gsem0, ssem0),
            (rowf1, colb1, wb1, gbuf1, gsem1, ssem1),
            (rowf2, colb2, wb2, gbuf2, gsem2, ssem2))

    def load_idx(bi, b):
        rowf, colb, wb, gbuf, gsem, ssem = bufs[bi]
        eb = e0 + b * _BLK
        pltpu.sync_copy(row_hbm.at[pl.ds(eb, _BLK)], rowf)
        pltpu.sync_copy(col_hbm.at[pl.ds(eb, _BLK)], colf)
        pltpu.sync_copy(w_hbm.at[pl.ds(eb, _BLK)], wb)
        _repack(colf, colb, _NSEG)

    def fire_gathers(bi):
        rowf, colb, wb, gbuf, gsem, ssem = bufs[bi]
        for j in range(_NSEG):
            pltpu.async_copy(g_hbm.at[rowf.at[pl.ds(j * _SEG, _SEG)]],
                             gbuf.at[pl.ds(j * _SEG, _SEG)], gsem)

    def drain_gathers(bi):
        rowf, colb, wb, gbuf, gsem, ssem = bufs[bi]
        for j in range(_NSEG):
            pltpu.make_async_copy(g_hbm.at[rowf.at[pl.ds(j * _SEG, _SEG)]],
                                  gbuf.at[pl.ds(j * _SEG, _SEG)], gsem).wait()

    def fire_scatters(bi):
        rowf, colb, wb, gbuf, gsem, ssem = bufs[bi]
        for j in range(_NSEG):
            pltpu.async_copy(gbuf.at[pl.ds(j * _SEG, _SEG)],
                             acc_sh.at[colb.at[j]], ssem, add=True)

    def drain_scatters(bi):
        rowf, colb, wb, gbuf, gsem, ssem = bufs[bi]
        for j in range(_NSEG):
            pltpu.make_async_copy(gbuf.at[pl.ds(j * _SEG, _SEG)],
                                  acc_sh.at[colb.at[j]], ssem).wait()

    def scale(bi):
        rowf, colb, wb, gbuf, gsem, ssem = bufs[bi]
        nq = _H // 16

        @pl.loop(0, _BLK, step=16)
        def _(i):
            wv = wb[pl.ds(i, 16)]
            for k0 in range(0, 16, 4):
                ws = [jnp.full((16,), wv[k0 + d], jnp.float32)
                      for d in range(4)]
                vals = [gbuf[i + k0 + d, pl.ds(q * 16, 16)]
                        for d in range(4) for q in range(nq)]
                outs = [vals[d * nq + q] * ws[d]
                        for d in range(4) for q in range(nq)]
                for d in range(4):
                    for q in range(nq):
                        gbuf[i + k0 + d, pl.ds(q * 16, 16)] = outs[d * nq + q]

    @pl.loop(0, _BLK)
    def _(r):
        for q in range(_H // 16):
            gbuf0[r, pl.ds(q * 16, 16)] = jnp.zeros((16,), jnp.float32)

    pltpu.sync_copy(gbuf0, acc_sh.at[pl.ds(s * _RPS, _BLK)])
    pltpu.sync_copy(gbuf0.at[pl.ds(0, _RPS - _BLK)],
                    acc_sh.at[pl.ds(s * _RPS + _BLK, _RPS - _BLK)])
    plsc.subcore_barrier()

    load_idx(0, 0)
    fire_gathers(0)
    load_idx(1, 1)
    fire_gathers(1)

    @pl.loop(0, _NBLK - 1, step=_NBUF)
    def _(b):
        for k in range(_NBUF):
            cur = k % _NBUF
            nxt = (k + 2) % _NBUF
            drain_gathers(cur)
            scale(cur)
            fire_scatters(cur)
            if k == 0:
                @pl.when(b > 0)
                def _():
                    drain_scatters(nxt)
            else:
                drain_scatters(nxt)
            if k == _NBUF - 1:
                @pl.when(b + k + 2 < _NBLK)
                def _():
                    load_idx(nxt, b + k + 2)
                    fire_gathers(nxt)
            else:
                load_idx(nxt, b + k + 2)
                fire_gathers(nxt)

    drain_gathers(0)
    scale(0)
    fire_scatters(0)
    drain_scatters(2)
    drain_scatters(0)
    plsc.subcore_barrier()
    pltpu.sync_copy(acc_sh.at[pl.ds(s * _RPS, _RPS)],
                    p_hbm.at[c].at[pl.ds(s * _RPS, _RPS)])


_mp = pl.kernel(
    _mp_body,
    out_type=jax.ShapeDtypeStruct((_NC, _N, _H), jnp.float32),
    mesh=_mesh,
    compiler_params=_sc_params,
    scratch_types=(
        [pltpu.VMEM((_BLK,), jnp.int32),
         pltpu.VMEM((_NSEG, _SEG), jnp.int32),
         pltpu.VMEM((_BLK,), jnp.float32),
         pltpu.VMEM((_BLK, _H), jnp.float32),
         ] * _NBUF
        + [pltpu.VMEM((_BLK,), jnp.int32)]
        + [pltpu.VMEM_SHARED((_N, _H), jnp.float32)]
        + [pltpu.SemaphoreType.DMA] * (2 * _NBUF)
    ),
)


def _prep_body(hist_ref, x_ref, w1_ref, dis_ref, g_ref):
    hist = hist_ref[...]
    deg = hist[0] + hist[1] + 1.0
    dis = lax.rsqrt(deg)
    dis_ref[...] = jnp.reshape(dis, (_N, 1))
    xw = lax.dot_general(x_ref[...], w1_ref[...], (((1,), (1,)), ((), ())),
                         preferred_element_type=jnp.float32)
    g_ref[...] = jnp.reshape(dis, (_N, 1)) * xw


_prep = pl.pallas_call(
    _prep_body,
    out_shape=(
        jax.ShapeDtypeStruct((_N, 1), jnp.float32),
        jax.ShapeDtypeStruct((_N, _H), jnp.float32),
    ),
)


def _layer_body(p_ref, g_ref, dis_ref, b_ref, w_ref, gn_ref):
    dis = dis_ref[...]
    agg = p_ref[0] + p_ref[1] + g_ref[...]
    h = jnp.maximum(dis * agg + b_ref[...], 0.0)
    hw = lax.dot_general(h, w_ref[...], (((1,), (1,)), ((), ())),
                         preferred_element_type=jnp.float32)
    gn_ref[...] = dis * hw


_layer = pl.pallas_call(
    _layer_body,
    out_shape=jax.ShapeDtypeStruct((_N, _H), jnp.float32),
)


def _final_body(p_ref, g_ref, dis_ref, b_ref, batch_ref, wc_ref, bc_ref,
                out_ref):
    dis = dis_ref[...]
    h = jnp.maximum(dis * (p_ref[0] + p_ref[1] + g_ref[...]) + b_ref[...],
                    0.0)
    onehot = (batch_ref[...] ==
              lax.broadcasted_iota(jnp.int32, (_B, _N), 0)).astype(jnp.float32)
    sums = lax.dot_general(onehot, h, (((1,), (0,)), ((), ())),
                           preferred_element_type=jnp.float32)
    cnt = jnp.sum(onehot, axis=1, keepdims=True)
    pooled = sums / jnp.maximum(cnt, 1.0)
    out = lax.dot_general(pooled, wc_ref[...], (((1,), (1,)), ((), ())),
                          preferred_element_type=jnp.float32)
    out_ref[...] = out + bc_ref[...]


_final = pl.pallas_call(
    _final_body,
    out_shape=jax.ShapeDtypeStruct((_B, _C), jnp.float32),
)


def kernel(x, edge_index, edge_attr, batch, W1, b1, W2, b2, W3, b3, Wc, bc):
    row = edge_index[0]
    col = edge_index[1]
    hist, w = _hist_w(col, edge_attr)
    dis, g1 = _prep(hist, x, W1)
    p1 = _mp(g1, row, col, w)
    g2 = _layer(p1, g1, dis, b1.reshape(1, _H), W2)
    p2 = _mp(g2, row, col, w)
    g3 = _layer(p2, g2, dis, b2.reshape(1, _H), W3)
    p3 = _mp(g3, row, col, w)
    return _final(p3, g3, dis, b3.reshape(1, _H), batch.reshape(1, _N),
                  Wc, bc.reshape(1, _C))
